# Initial kernel scaffold; baseline (speedup 1.0000x reference)
import jax
import jax.numpy as jnp
from jax.experimental import pallas as pl


def kernel(input_ids, attention_mask, emb_table, W, b):
    raise NotImplementedError("write your pallas kernel here")


# speedup vs baseline: 14.9401x; 14.9401x over previous
import functools

import jax
import jax.numpy as jnp
from jax import lax
from jax.experimental import pallas as pl
from jax.experimental.pallas import tpu as pltpu
from jax.experimental.pallas import tpu_sc as plsc

VOCAB = 30522
HIDDEN = 768
NUM_LABELS = 2
BATCH = 4096
SEQ = 200

L = 16
NC = 2
NS = 16
NW = NC * NS
ROWS_PER_W = BATCH // NW
SEQP = 208
G_ROWS = 8
NG = ROWS_PER_W // G_ROWS
TOK_G = G_ROWS * SEQP
N_GATHER = TOK_G // 128
ZROW = VOCAB
VP = VOCAB + 6

_PROJ_BLK = 512


def _proj_body(emb_ref, w_ref, out_ref):
    res = jnp.dot(emb_ref[...], w_ref[...],
                  preferred_element_type=jnp.float32)
    col = lax.broadcasted_iota(jnp.int32, res.shape, 1)
    out_ref[...] = jnp.where(col == NUM_LABELS, 1.0, res)


def _project_table(emb_table, w_pad):
    grid = (pl.cdiv(VOCAB, _PROJ_BLK),)
    return pl.pallas_call(
        _proj_body,
        grid=grid,
        in_specs=[
            pl.BlockSpec((_PROJ_BLK, HIDDEN), lambda i: (i, 0)),
            pl.BlockSpec((HIDDEN, L), lambda i: (0, 0)),
        ],
        out_specs=pl.BlockSpec((_PROJ_BLK, L), lambda i: (i, 0)),
        out_shape=jax.ShapeDtypeStruct((VOCAB, L), jnp.float32),
    )(emb_table, w_pad)


def _sc_body(ids_hbm, mask_hbm, proj_hbm, bias_hbm, out_hbm,
             idsv, maskv, idx2d, rowsv, outv, biasv, sem):
    wid = lax.axis_index("s") * NC + lax.axis_index("c")
    base_row = wid * ROWS_PER_W

    pltpu.sync_copy(bias_hbm, biasv)
    bias_vec = biasv[...]

    def group_body(g, _):
        row0 = base_row + g * G_ROWS
        tok0 = row0 * SEQP
        pltpu.sync_copy(ids_hbm.at[pl.ds(tok0, TOK_G)], idsv)
        pltpu.sync_copy(mask_hbm.at[pl.ds(tok0, TOK_G)], maskv)

        def sel_body(k, _):
            ids16 = idsv[pl.ds(k * 16, 16)]
            m16 = maskv[pl.ds(k * 16, 16)]
            sel = jnp.where(m16 != 0, ids16, ZROW)
            j = k // 8
            l16 = (k - j * 8) * 16
            idx2d[j, pl.ds(l16, 16)] = sel
            return 0
        lax.fori_loop(0, TOK_G // 16, sel_body, 0)

        descs = [
            pltpu.async_copy(proj_hbm.at[idx2d.at[j]],
                             rowsv.at[pl.ds(j * 128, 128)], sem)
            for j in range(N_GATHER)
        ]
        for d in descs:
            d.wait()

        def row_body(r, _):
            tb = r * SEQP

            def acc_body(c, accs):
                a0, a1, a2, a3 = accs
                t = tb + c * 8
                a0 = a0 + rowsv[t + 0] + rowsv[t + 4]
                a1 = a1 + rowsv[t + 1] + rowsv[t + 5]
                a2 = a2 + rowsv[t + 2] + rowsv[t + 6]
                a3 = a3 + rowsv[t + 3] + rowsv[t + 7]
                return (a0, a1, a2, a3)
            zero = jnp.zeros((16,), jnp.float32)
            a0, a1, a2, a3 = lax.fori_loop(0, SEQP // 8, acc_body,
                                           (zero, zero, zero, zero))
            total = (a0 + a1) + (a2 + a3)
            idx = jnp.full((16, 1), NUM_LABELS, jnp.int32)
            dnums = lax.GatherDimensionNumbers(
                offset_dims=(), collapsed_slice_dims=(0,),
                start_index_map=(0,))
            cnt_vec = lax.gather(
                total, idx, dnums, slice_sizes=(1,),
                mode=lax.GatherScatterMode.PROMISE_IN_BOUNDS)
            outv[r] = total / cnt_vec + bias_vec
            return 0
        lax.fori_loop(0, G_ROWS, row_body, 0)

        pltpu.sync_copy(outv, out_hbm.at[pl.ds(row0, G_ROWS)])
        return 0

    lax.fori_loop(0, NG, group_body, 0)


def _sc_pool(ids_flat, mask_flat, proj_pad, bias_pad):
    mesh = plsc.VectorSubcoreMesh(core_axis_name="c", subcore_axis_name="s",
                                  num_cores=NC, num_subcores=NS)
    f = pl.kernel(
        _sc_body,
        out_type=jax.ShapeDtypeStruct((BATCH, L), jnp.float32),
        mesh=mesh,
        scratch_types=[
            pltpu.VMEM((TOK_G,), jnp.int32),
            pltpu.VMEM((TOK_G,), jnp.int32),
            pltpu.VMEM((N_GATHER, 128), jnp.int32),
            pltpu.VMEM((TOK_G, L), jnp.float32),
            pltpu.VMEM((G_ROWS, L), jnp.float32),
            pltpu.VMEM((L,), jnp.float32),
            pltpu.SemaphoreType.DMA,
        ],
        compiler_params=pltpu.CompilerParams(use_tc_tiling_on_sc=False),
    )
    return f(ids_flat, mask_flat, proj_pad, bias_pad)


@jax.jit
def kernel(input_ids, attention_mask, emb_table, W, b):
    w_pad = jnp.pad(W.astype(jnp.float32), ((0, 0), (0, L - NUM_LABELS)))
    proj = _project_table(emb_table.astype(jnp.float32), w_pad)
    proj_pad = jnp.pad(proj, ((0, VP - VOCAB), (0, 0)))
    bias_pad = jnp.pad(b.astype(jnp.float32), (0, L - NUM_LABELS))

    ids_p = jnp.pad(input_ids.astype(jnp.int32), ((0, 0), (0, SEQP - SEQ)))
    mask_p = jnp.pad(attention_mask.astype(jnp.int32),
                     ((0, 0), (0, SEQP - SEQ)))
    pooled = _sc_pool(ids_p.reshape(-1), mask_p.reshape(-1),
                      proj_pad, bias_pad)
    return pooled[:, :NUM_LABELS]


# speedup vs baseline: 41.6179x; 2.7857x over previous
import jax
import jax.numpy as jnp
from jax import lax
from jax.experimental import pallas as pl
from jax.experimental.pallas import tpu as pltpu
from jax.experimental.pallas import tpu_sc as plsc

VOCAB = 30522
HIDDEN = 768
NUM_LABELS = 2
BATCH = 4096
SEQ = 200

L = 16
NC = 2
NS = 16
NW = NC * NS
ROWS_PER_W = BATCH // NW
LG = ROWS_PER_W // L
SEQP = 208
VP = VOCAB + 6
ZCOL = VOCAB
TROWS = 3
CHUNK = 13
NCHUNK = SEQP // CHUNK

_PROJ_BLK = 512


def _proj_body(w_ref, emb_ref, out_ref):
    res = lax.dot_general(w_ref[...], emb_ref[...],
                          (((0,), (1,)), ((), ())),
                          preferred_element_type=jnp.float32)
    row = lax.broadcasted_iota(jnp.int32, res.shape, 0)
    out_ref[...] = jnp.where(row == NUM_LABELS, 1.0, res)


def _project_table(emb_table, w_pad):
    grid = (pl.cdiv(VOCAB, _PROJ_BLK),)
    return pl.pallas_call(
        _proj_body,
        grid=grid,
        in_specs=[
            pl.BlockSpec((HIDDEN, TROWS), lambda i: (0, 0)),
            pl.BlockSpec((_PROJ_BLK, HIDDEN), lambda i: (i, 0)),
        ],
        out_specs=pl.BlockSpec((TROWS, _PROJ_BLK), lambda i: (0, i)),
        out_shape=jax.ShapeDtypeStruct((TROWS, VOCAB), jnp.float32),
    )(w_pad, emb_table)


def _lane_bcast(vec, lane):
    idx = jnp.full((L, 1), lane, jnp.int32)
    dnums = lax.GatherDimensionNumbers(
        offset_dims=(), collapsed_slice_dims=(0,), start_index_map=(0,))
    return lax.gather(vec, idx, dnums, slice_sizes=(1,),
                      mode=lax.GatherScatterMode.PROMISE_IN_BOUNDS)


def _sc_body(ids_hbm, mask_hbm, tab_hbm, bias_hbm, out_hbm,
             tabv, idsv, maskv, outv, biasv, sem_tab, sem_io):
    wid = lax.axis_index("s") * NC + lax.axis_index("c")

    tab_dma = pltpu.async_copy(tab_hbm, tabv, sem_tab)
    pltpu.sync_copy(bias_hbm, biasv)
    bias_vec = biasv[...]
    b0 = _lane_bcast(bias_vec, 0)
    b1 = _lane_bcast(bias_vec, 1)

    def fire(c, buf):
        pltpu.async_copy(ids_hbm.at[wid, pl.ds(c * CHUNK, CHUNK)],
                         idsv.at[buf], sem_io)
        pltpu.async_copy(mask_hbm.at[wid, pl.ds(c * CHUNK, CHUNK)],
                         maskv.at[buf], sem_io)

    def drain(buf):
        pltpu.make_async_copy(ids_hbm.at[0, pl.ds(0, CHUNK)],
                              idsv.at[buf], sem_io).wait()
        pltpu.make_async_copy(mask_hbm.at[0, pl.ds(0, CHUNK)],
                              maskv.at[buf], sem_io).wait()

    fire(0, 0)
    tab_dma.wait()

    c1 = jnp.full((L,), VP, jnp.int32)
    c2 = jnp.full((L,), 2 * VP, jnp.int32)
    zcol = jnp.full((L,), ZCOL, jnp.int32)

    def consume(buf, accs):
        accs = list(accs)
        for si in range(CHUNK):
            for lg in range(LG):
                ids16 = idsv[buf, si, pl.ds(lg * L, L)]
                m16 = maskv[buf, si, pl.ds(lg * L, L)]
                sel = jnp.where(m16 != 0, ids16, zcol)
                g0 = plsc.load_gather(tabv, [sel])
                g1 = plsc.load_gather(tabv, [sel + c1])
                g2 = plsc.load_gather(tabv, [sel + c2])
                a0, a1, a2 = accs[3 * lg:3 * lg + 3]
                accs[3 * lg:3 * lg + 3] = [a0 + g0, a1 + g1, a2 + g2]
        return tuple(accs)

    def pair_body(p, accs):
        c = 2 * p

        @pl.when(c + 1 < NCHUNK)
        def _():
            fire(c + 1, 1)
        drain(0)
        accs = consume(0, accs)

        @pl.when(c + 2 < NCHUNK)
        def _():
            fire(c + 2, 0)

        @pl.when(c + 1 < NCHUNK)
        def _():
            drain(1)
        accs = lax.cond(c + 1 < NCHUNK, lambda a: consume(1, a),
                        lambda a: a, accs)
        return accs

    zero = jnp.zeros((L,), jnp.float32)
    accs = lax.fori_loop(0, (NCHUNK + 1) // 2, pair_body,
                         tuple(zero for _ in range(3 * LG)))

    for lg in range(LG):
        a0, a1, a2 = accs[3 * lg:3 * lg + 3]
        outv[0, pl.ds(lg * L, L)] = a0 / a2 + b0
        outv[1, pl.ds(lg * L, L)] = a1 / a2 + b1
    pltpu.sync_copy(outv, out_hbm.at[wid])


def _sc_pool(ids3, mask3, tab_flat, bias_pad):
    mesh = plsc.VectorSubcoreMesh(core_axis_name="c", subcore_axis_name="s",
                                  num_cores=NC, num_subcores=NS)
    f = pl.kernel(
        _sc_body,
        out_type=jax.ShapeDtypeStruct((NW, NUM_LABELS, ROWS_PER_W),
                                      jnp.float32),
        mesh=mesh,
        scratch_types=[
            pltpu.VMEM((TROWS * VP,), jnp.float32),
            pltpu.VMEM((2, CHUNK, ROWS_PER_W), jnp.int32),
            pltpu.VMEM((2, CHUNK, ROWS_PER_W), jnp.int32),
            pltpu.VMEM((NUM_LABELS, ROWS_PER_W), jnp.float32),
            pltpu.VMEM((L,), jnp.float32),
            pltpu.SemaphoreType.DMA,
            pltpu.SemaphoreType.DMA,
        ],
        compiler_params=pltpu.CompilerParams(use_tc_tiling_on_sc=False,
                                             needs_layout_passes=False),
    )
    return f(ids3, mask3, tab_flat, bias_pad)


@jax.jit
def kernel(input_ids, attention_mask, emb_table, W, b):
    w_pad = jnp.pad(W.astype(jnp.float32), ((0, 0), (0, TROWS - NUM_LABELS)))
    proj = _project_table(emb_table.astype(jnp.float32), w_pad)
    tab_flat = jnp.pad(proj, ((0, 0), (0, VP - VOCAB))).reshape(-1)
    bias_pad = jnp.pad(b.astype(jnp.float32), (0, L - NUM_LABELS))

    ids_p = jnp.pad(input_ids.astype(jnp.int32), ((0, 0), (0, SEQP - SEQ)))
    mask_p = jnp.pad(attention_mask.astype(jnp.int32),
                     ((0, 0), (0, SEQP - SEQ)))
    ids3 = ids_p.reshape(NW, ROWS_PER_W, SEQP).transpose(0, 2, 1)
    mask3 = mask_p.reshape(NW, ROWS_PER_W, SEQP).transpose(0, 2, 1)

    out3 = _sc_pool(ids3, mask3, tab_flat, bias_pad)
    return out3.transpose(0, 2, 1).reshape(BATCH, NUM_LABELS)
